# speedup vs baseline: 17.1564x; 17.1564x over previous
import functools

import jax
import jax.numpy as jnp
from jax import lax
from jax.experimental import pallas as pl
from jax.experimental.pallas import tpu as pltpu
from jax.experimental.pallas import tpu_sc as plsc

VOCAB = 1000000
EMBED = 128
SEQ = 200
BATCH = 4096


_TBLK = 4000


def _transform_body(tab_ref, w_ref, b_ref, out_ref):
    x = tab_ref[...]
    out_ref[...] = (
        lax.dot_general(
            x, w_ref[...], (((1,), (1,)), ((), ())),
            preferred_element_type=jnp.float32,
        )
        + b_ref[...]
    )


def _transform_table(table, W, b):
    return pl.pallas_call(
        _transform_body,
        grid=(VOCAB // _TBLK,),
        in_specs=[
            pl.BlockSpec((_TBLK, EMBED), lambda i: (i, 0)),
            pl.BlockSpec((EMBED, EMBED), lambda i: (0, 0)),
            pl.BlockSpec((1, EMBED), lambda i: (0, 0)),
        ],
        out_specs=pl.BlockSpec((_TBLK, EMBED), lambda i: (i, 0)),
        out_shape=jax.ShapeDtypeStruct((VOCAB, EMBED), jnp.float32),
    )(table, W, b.reshape(1, EMBED))


_NC = 2
_NS = 16
_NW = _NC * _NS
_COLS_PER_W = BATCH // _NW
_TOK_PER_W = _COLS_PER_W * SEQ
_CH0 = 104
_CH1 = SEQ - _CH0
_NVR = EMBED // 16


@functools.partial(
    pl.kernel,
    out_type=jax.ShapeDtypeStruct((BATCH, EMBED), jnp.float32),
    mesh=plsc.VectorSubcoreMesh(core_axis_name="c", subcore_axis_name="s"),
    scratch_types=[
        pltpu.VMEM((_TOK_PER_W,), jnp.int32),
        pltpu.VMEM((_CH0, EMBED), jnp.float32),
        pltpu.VMEM((_CH1, EMBED), jnp.float32),
        pltpu.VMEM((_COLS_PER_W, EMBED), jnp.float32),
        pltpu.SemaphoreType.DMA,
        pltpu.SemaphoreType.DMA,
    ],
)
def _gather_max(tp_hbm, idx_hbm, out_hbm, idx_v, buf0, buf1, acc_v, sem0, sem1):
    wid = lax.axis_index("s") * _NC + lax.axis_index("c")

    tok_base = pl.multiple_of(wid * _TOK_PER_W, 8)
    pltpu.sync_copy(idx_hbm.at[pl.ds(tok_base, _TOK_PER_W)], idx_v)

    def col_body(c, carry):
        base = pl.multiple_of(c * SEQ, 8)
        cp0 = pltpu.async_copy(
            tp_hbm.at[idx_v.at[pl.ds(base, _CH0)]], buf0, sem0)
        cp1 = pltpu.async_copy(
            tp_hbm.at[idx_v.at[pl.ds(base + _CH0, _CH1)]], buf1, sem1)
        cp0.wait()
        cp1.wait()

        def red0(s, acc):
            return tuple(
                jnp.maximum(acc[k], buf0[s, pl.ds(16 * k, 16)])
                for k in range(_NVR))

        def red1(s, acc):
            return tuple(
                jnp.maximum(acc[k], buf1[s, pl.ds(16 * k, 16)])
                for k in range(_NVR))

        neg = jnp.full((16,), -jnp.inf, jnp.float32)
        acc = tuple(neg for _ in range(_NVR))
        acc = lax.fori_loop(0, _CH0, red0, acc)
        acc = lax.fori_loop(0, _CH1, red1, acc)
        for k in range(_NVR):
            acc_v[c, pl.ds(16 * k, 16)] = acc[k]
        return carry

    lax.fori_loop(0, _COLS_PER_W, col_body, 0)

    col_base = pl.multiple_of(wid * _COLS_PER_W, 8)
    pltpu.sync_copy(acc_v, out_hbm.at[pl.ds(col_base, _COLS_PER_W)])


def kernel(inputs, table, W, b):
    tp = _transform_table(table, W, b)
    idx = jnp.asarray(inputs, jnp.int32).T.reshape(-1)
    return _gather_max(tp, idx)
